# Initial kernel scaffold; baseline (speedup 1.0000x reference)
import jax
import jax.numpy as jnp
from jax.experimental import pallas as pl


def kernel(feat, edge_index, iw, ow, bn_gamma, bn_beta, W_in, b_in, W_out, b_out, Wg_in, bg_in, Wg_out):
    raise NotImplementedError("write your pallas kernel here")


# speedup vs baseline: 4.7983x; 4.7983x over previous
import dataclasses

import jax
import jax.numpy as jnp
from jax import lax
from jax.experimental import pallas as pl
from jax.experimental.pallas import tpu as pltpu
from jax.experimental.pallas import tpu_sc as plsc

_N = 10000
_E = 320000
_H = 128
_NSUB = 16
_CHUNK = 128
_CHUNKS = 157
_EPAD = _NSUB * _CHUNKS * _CHUNK
_ROWS_PER_TILE = 624
_ROWS_REM = _N - _NSUB * _ROWS_PER_TILE


def _dense_in_body(feat_ref, g_ref, b_ref, wi_ref, bi_ref, wo_ref, bo_ref,
                   x_ref, t_ref):
    f = feat_ref[...]
    mean = jnp.mean(f, axis=0, keepdims=True)
    cen = f - mean
    var = jnp.mean(cen * cen, axis=0, keepdims=True)
    x = cen * lax.rsqrt(var + 1e-5) * g_ref[...] + b_ref[...]
    x_ref[...] = x
    cd = (((1,), (1,)), ((), ()))
    t_ref[0:_N, :] = lax.dot_general(
        x, wi_ref[...], cd, preferred_element_type=jnp.float32) + bi_ref[...]
    t_ref[_N:, :] = lax.dot_general(
        x, wo_ref[...], cd, preferred_element_type=jnp.float32) + bo_ref[...]


def _dense_out_body(x_ref, a_ref, wg_ref, bg_ref, wo_ref, o_ref):
    cd = (((1,), (1,)), ((), ()))
    x = x_ref[...]
    wg = wg_ref[...]
    f = (lax.dot_general(a_ref[0], wg[:, :_H], cd,
                         preferred_element_type=jnp.float32)
         + lax.dot_general(a_ref[1], wg[:, _H:], cd,
                           preferred_element_type=jnp.float32)
         + bg_ref[...])
    s = f + lax.dot_general(x, wo_ref[...], cd,
                            preferred_element_type=jnp.float32)
    ig = jax.nn.sigmoid(s[:, :_H])
    ng = jnp.tanh(s[:, _H:])
    o_ref[...] = ng + ig * (x - ng)


def _sc_edge_body(table_ref, edata_ref, out_ref, ebuf, rows_v, acc):
    c = lax.axis_index("c")
    s = lax.axis_index("s")
    zeros16 = jnp.zeros((16,), jnp.float32)

    @pl.loop(0, _CHUNK)
    def _(i):
        for k in range(_H // 16):
            rows_v[i, pl.ds(16 * k, 16)] = zeros16

    base = s * _ROWS_PER_TILE
    off = 0
    while off < _ROWS_PER_TILE:
        sz = min(_CHUNK, _ROWS_PER_TILE - off)
        pltpu.sync_copy(rows_v.at[pl.ds(0, sz)], acc.at[pl.ds(base + off, sz)])
        off += sz

    @pl.when(s == 0)
    def _():
        pltpu.sync_copy(rows_v.at[pl.ds(0, _ROWS_REM)],
                        acc.at[pl.ds(_NSUB * _ROWS_PER_TILE, _ROWS_REM)])

    plsc.subcore_barrier()

    @pl.loop(0, _CHUNKS)
    def _(j):
        pltpu.sync_copy(edata_ref.at[c, s, j], ebuf)

        pltpu.sync_copy(table_ref.at[ebuf.at[0]], rows_v)

        @pl.loop(0, _CHUNK, step=16)
        def _(g):
            wv = plsc.bitcast(ebuf[2, pl.ds(g, 16)], jnp.float32)
            for t in range(16):
                w = wv[t]
                for k in range(_H // 16):
                    sl = pl.ds(16 * k, 16)
                    rows_v[g + t, sl] = rows_v[g + t, sl] * w

        pltpu.sync_copy(rows_v, acc.at[ebuf.at[1]], add=True)

    plsc.subcore_barrier()
    pltpu.sync_copy(acc.at[pl.ds(base, _ROWS_PER_TILE)],
                    out_ref.at[c, pl.ds(base, _ROWS_PER_TILE)])

    @pl.when(s == 0)
    def _():
        pltpu.sync_copy(acc.at[pl.ds(_NSUB * _ROWS_PER_TILE, _ROWS_REM)],
                        out_ref.at[c, pl.ds(_NSUB * _ROWS_PER_TILE, _ROWS_REM)])


def kernel(feat, edge_index, iw, ow, bn_gamma, bn_beta, W_in, b_in,
           W_out, b_out, Wg_in, bg_in, Wg_out):
    f32 = jnp.float32
    x, table = pl.pallas_call(
        _dense_in_body,
        out_shape=(jax.ShapeDtypeStruct((_N, _H), f32),
                   jax.ShapeDtypeStruct((2 * _N, _H), f32)),
    )(feat, bn_gamma.reshape(1, _H), bn_beta.reshape(1, _H),
      W_in, b_in.reshape(1, _H), W_out, b_out.reshape(1, _H))

    src = edge_index[0]
    dst = edge_index[1]
    pad = _EPAD - _E
    zi = jnp.zeros((pad,), jnp.int32)
    zf = jnp.zeros((pad,), f32)
    src_p = jnp.concatenate([src, zi])
    dst_p = jnp.concatenate([dst, zi])
    gidx = jnp.stack([src_p, dst_p + _N]).reshape(2, _NSUB, _CHUNKS, _CHUNK)
    sidx = jnp.stack([dst_p, src_p]).reshape(2, _NSUB, _CHUNKS, _CHUNK)
    warr = jnp.stack([jnp.concatenate([iw[:, 0], zf]),
                      jnp.concatenate([ow[:, 0], zf])]
                     ).reshape(2, _NSUB, _CHUNKS, _CHUNK)
    edata = jnp.stack(
        [gidx, sidx, lax.bitcast_convert_type(warr, jnp.int32)], axis=3)

    mesh = plsc.VectorSubcoreMesh(core_axis_name="c", subcore_axis_name="s")
    cp = pltpu.CompilerParams()
    if "needs_layout_passes" in pltpu.CompilerParams.__dataclass_fields__:
        cp = dataclasses.replace(cp, needs_layout_passes=False)
    a_io = pl.kernel(
        _sc_edge_body,
        mesh=mesh,
        compiler_params=cp,
        out_type=jax.ShapeDtypeStruct((2, _N, _H), f32),
        scratch_types=[
            pltpu.VMEM((3, _CHUNK), jnp.int32),
            pltpu.VMEM((_CHUNK, _H), f32),
            pltpu.VMEM_SHARED((_N, _H), f32),
        ],
    )(table, edata)

    hn = pl.pallas_call(
        _dense_out_body,
        out_shape=jax.ShapeDtypeStruct((_N, _H), f32),
    )(x, a_io, Wg_in, bg_in.reshape(1, 2 * _H), Wg_out)
    return hn
